# baseline (speedup 1.0000x reference)
import jax
import jax.numpy as jnp
from jax import lax
from jax.experimental import pallas as pl
from jax.experimental.pallas import tpu as pltpu
from jax.experimental.pallas import tpu_sc as plsc

B_, H_, W_, CH = 4, 128, 128, 96
N = B_ * H_ * W_ * CH
TOTAL = N * 4
NC, NS, L = 2, 16, 16

C = 1_572_864
NPASS = 8
S_TILE = C // NS
BLK = 4_096
PER_TILE = N // NS
NBLK = PER_TILE // BLK
NSTRM = 4
Q = 512
RQ = Q + L


def _body(upd_hbm, idx_hbm, out_hbm, acc, idxb, valb,
          lq0, lq1, lq2, lq3, vq0, vq1, vq2, vq3, posr):
    locq = [lq0, lq1, lq2, lq3]
    valq = [vq0, vq1, vq2, vq3]
    core = lax.axis_index("c")
    sub = lax.axis_index("s")
    tile_start = sub * PER_TILE
    zeros16 = jnp.zeros((L,), jnp.float32)
    iota16 = lax.iota(jnp.int32, L)

    def _zero_vals(s):
        for i in range(RQ // L + 1):
            off = min(i * L, RQ - L)
            valq[s][pl.ds(off, L)] = zeros16

    def _flush(s):
        pltpu.sync_copy(valq[s], acc.at[locq[s]], add=True)

    for p in range(NPASS):
        chunk = 2 * p + core
        base = chunk * C

        def _z(i, _):
            valb[pl.ds(i * L, L)] = zeros16
            return 0
        lax.fori_loop(0, BLK // L, _z, 0)
        for j in range(S_TILE // BLK):
            pltpu.sync_copy(valb, acc.at[pl.ds(sub * S_TILE + j * BLK, BLK)])

        for s in range(NSTRM):
            _zero_vals(s)
            for i in range(RQ // L + 1):
                off = min(i * L, RQ - L)
                locq[s][pl.ds(off, L)] = iota16 * 61 + s
            posr[s] = 0
        plsc.subcore_barrier()

        def _blk(b, _):
            st = tile_start + b * BLK
            pltpu.sync_copy(idx_hbm.at[pl.ds(st, BLK)], idxb)
            pltpu.sync_copy(upd_hbm.at[pl.ds(st, BLK)], valb)

            def _vec(i, _):
                for s in range(NSTRM):
                    off = i * (NSTRM * L) + s * L
                    vi = idxb[pl.ds(off, L)]
                    vv = valb[pl.ds(off, L)]
                    local = vi - base
                    inr = plsc.bitcast(local, jnp.uint32) < jnp.uint32(C)
                    m32 = jnp.where(inr, 1, 0)
                    pos = posr[s]
                    tgt = (pos - 1) + plsc.cumsum(m32)
                    plsc.store_scatter(locq[s], [tgt], local, mask=inr)
                    plsc.store_scatter(valq[s], [tgt], vv, mask=inr)
                    pos = pos + jnp.sum(m32)
                    posr[s] = pos

                    @pl.when(pos > Q - L)
                    def _():
                        _flush(s)
                        _zero_vals(s)
                        posr[s] = 0
                return 0
            lax.fori_loop(0, BLK // (NSTRM * L), _vec, 0)
            return 0
        lax.fori_loop(0, NBLK, _blk, 0)

        for s in range(NSTRM):
            _flush(s)
        plsc.subcore_barrier()

        pltpu.sync_copy(acc.at[pl.ds(sub * S_TILE, S_TILE)],
                        out_hbm.at[pl.ds(base + sub * S_TILE, S_TILE)])
        plsc.subcore_barrier()


_scatter = pl.kernel(
    _body,
    out_type=jax.ShapeDtypeStruct((TOTAL,), jnp.float32),
    mesh=plsc.VectorSubcoreMesh(
        core_axis_name="c", subcore_axis_name="s", num_cores=NC,
        num_subcores=NS),
    compiler_params=pltpu.CompilerParams(needs_layout_passes=False),
    scratch_types=[
        pltpu.VMEM_SHARED((C,), jnp.float32),
        pltpu.VMEM((BLK,), jnp.int32),
        pltpu.VMEM((BLK,), jnp.float32),
        pltpu.VMEM((RQ,), jnp.int32),
        pltpu.VMEM((RQ,), jnp.int32),
        pltpu.VMEM((RQ,), jnp.int32),
        pltpu.VMEM((RQ,), jnp.int32),
        pltpu.VMEM((RQ,), jnp.float32),
        pltpu.VMEM((RQ,), jnp.float32),
        pltpu.VMEM((RQ,), jnp.float32),
        pltpu.VMEM((RQ,), jnp.float32),
        pltpu.SMEM((NSTRM,), jnp.int32),
    ],
)


@jax.jit
def kernel(updates, mask):
    upd = updates.reshape(-1)
    idx = mask.reshape(-1).astype(jnp.int32)
    out = _scatter(upd, idx)
    return out.reshape(B_, H_ * 2, W_ * 2, CH)


# speedup vs baseline: 7.1088x; 7.1088x over previous
import jax
import jax.numpy as jnp
from jax import lax
from jax.experimental import pallas as pl
from jax.experimental.pallas import tpu as pltpu
from jax.experimental.pallas import tpu_sc as plsc

B_, H_, W_, CH = 4, 128, 128, 96
N = B_ * H_ * W_ * CH
TOTAL = N * 4
NC, NS, L = 2, 16, 16

C = 1_572_864
NPASS = 8
S_TILE = C // NS
BLK = 3_072
PER_TILE = N // NS
NBLK = PER_TILE // BLK
DUMMY_MASK = (1 << 20) - 1


def _body(upd_hbm, idx_hbm, out_hbm, acc,
          ib0, ib1, vb0, vb1, lb0, lb1, ob0, ob1,
          sin0, sin1, ssc0, ssc1):
    idxb = [ib0, ib1]
    valb = [vb0, vb1]
    locb = [lb0, lb1]
    vout = [ob0, ob1]
    sin = [sin0, sin1]
    ssc = [ssc0, ssc1]
    core = lax.axis_index("c")
    sub = lax.axis_index("s")
    tile_start = sub * PER_TILE
    zeros16 = jnp.zeros((L,), jnp.float32)

    def _issue_in(b, par):
        st = tile_start + b * BLK
        pltpu.async_copy(idx_hbm.at[pl.ds(st, BLK)], idxb[par], sin[par])
        pltpu.async_copy(upd_hbm.at[pl.ds(st, BLK)], valb[par], sin[par])

    def _wait_in(b, par):
        st = tile_start + b * BLK
        pltpu.make_async_copy(idx_hbm.at[pl.ds(st, BLK)], idxb[par],
                              sin[par]).wait()
        pltpu.make_async_copy(upd_hbm.at[pl.ds(st, BLK)], valb[par],
                              sin[par]).wait()

    def _wait_scat(par):
        pltpu.make_async_copy(vout[par], acc.at[locb[par]], ssc[par]).wait()

    for p in range(NPASS):
        chunk = 2 * p + core
        base = chunk * C

        def _z(i, _):
            ob0[pl.ds(i * L, L)] = zeros16
            return 0
        lax.fori_loop(0, BLK // L, _z, 0)
        for j in range(S_TILE // BLK):
            pltpu.sync_copy(ob0, acc.at[pl.ds(sub * S_TILE + j * BLK, BLK)])
        plsc.subcore_barrier()

        _issue_in(0, 0)
        _issue_in(1, 1)

        def _blk2(g, _):
            for par in range(2):
                b = g * 2 + par
                _wait_in(b, par)

                @pl.when(b >= 2)
                def _():
                    _wait_scat(par)

                def _vec(i, _):
                    off = i * L
                    vi = idxb[par][pl.ds(off, L)]
                    vv = valb[par][pl.ds(off, L)]
                    local = vi - base
                    inr = plsc.bitcast(local, jnp.uint32) < jnp.uint32(C)
                    dummy = vi & DUMMY_MASK
                    locb[par][pl.ds(off, L)] = jnp.where(inr, local, dummy)
                    vout[par][pl.ds(off, L)] = jnp.where(inr, vv, 0.0)
                    return 0
                lax.fori_loop(0, BLK // L, _vec, 0)

                pltpu.async_copy(vout[par], acc.at[locb[par]], ssc[par],
                                 add=True)

                @pl.when(b + 2 < NBLK)
                def _():
                    _issue_in(b + 2, par)
            return 0
        lax.fori_loop(0, NBLK // 2, _blk2, 0)

        _wait_scat(0)
        _wait_scat(1)
        plsc.subcore_barrier()

        pltpu.sync_copy(acc.at[pl.ds(sub * S_TILE, S_TILE)],
                        out_hbm.at[pl.ds(base + sub * S_TILE, S_TILE)])
        plsc.subcore_barrier()


_scatter = pl.kernel(
    _body,
    out_type=jax.ShapeDtypeStruct((TOTAL,), jnp.float32),
    mesh=plsc.VectorSubcoreMesh(
        core_axis_name="c", subcore_axis_name="s", num_cores=NC,
        num_subcores=NS),
    compiler_params=pltpu.CompilerParams(needs_layout_passes=False),
    scratch_types=[
        pltpu.VMEM_SHARED((C,), jnp.float32),
        pltpu.VMEM((BLK,), jnp.int32),
        pltpu.VMEM((BLK,), jnp.int32),
        pltpu.VMEM((BLK,), jnp.float32),
        pltpu.VMEM((BLK,), jnp.float32),
        pltpu.VMEM((BLK,), jnp.int32),
        pltpu.VMEM((BLK,), jnp.int32),
        pltpu.VMEM((BLK,), jnp.float32),
        pltpu.VMEM((BLK,), jnp.float32),
        pltpu.SemaphoreType.DMA,
        pltpu.SemaphoreType.DMA,
        pltpu.SemaphoreType.DMA,
        pltpu.SemaphoreType.DMA,
    ],
)


@jax.jit
def kernel(updates, mask):
    upd = updates.reshape(-1)
    idx = mask.reshape(-1).astype(jnp.int32)
    out = _scatter(upd, idx)
    return out.reshape(B_, H_ * 2, W_ * 2, CH)


# speedup vs baseline: 7.2004x; 1.0129x over previous
import jax
import jax.numpy as jnp
from jax import lax
from jax.experimental import pallas as pl
from jax.experimental.pallas import tpu as pltpu
from jax.experimental.pallas import tpu_sc as plsc

B_, H_, W_, CH = 4, 128, 128, 96
N = B_ * H_ * W_ * CH
TOTAL = N * 4
NC, NS, L = 2, 16, 16

C = 1_572_864
NPASS = 8
S_TILE = C // NS
BLK = 3_072
PER_TILE = N // NS
NBLK = PER_TILE // BLK
DUMMY_MASK = (1 << 20) - 1


def _body(upd_hbm, idx_hbm, out_hbm, acc,
          ib0, ib1, vb0, vb1, lb0, lb1, ob0, ob1,
          sin0, sin1, ssc0, ssc1):
    idxb = [ib0, ib1]
    valb = [vb0, vb1]
    locb = [lb0, lb1]
    vout = [ob0, ob1]
    sin = [sin0, sin1]
    ssc = [ssc0, ssc1]
    core = lax.axis_index("c")
    sub = lax.axis_index("s")
    tile_start = sub * PER_TILE
    zeros16 = jnp.zeros((L,), jnp.float32)

    def _issue_in(b, par):
        st = tile_start + b * BLK
        pltpu.async_copy(idx_hbm.at[pl.ds(st, BLK)], idxb[par], sin[par])
        pltpu.async_copy(upd_hbm.at[pl.ds(st, BLK)], valb[par], sin[par])

    def _wait_in(b, par):
        st = tile_start + b * BLK
        pltpu.make_async_copy(idx_hbm.at[pl.ds(st, BLK)], idxb[par],
                              sin[par]).wait()
        pltpu.make_async_copy(upd_hbm.at[pl.ds(st, BLK)], valb[par],
                              sin[par]).wait()

    def _wait_scat(par):
        pltpu.make_async_copy(vout[par], acc.at[locb[par]], ssc[par]).wait()

    for p in range(NPASS):
        chunk = 2 * p + core
        base = chunk * C

        _issue_in(0, 0)
        _issue_in(1, 1)

        def _z(i, _):
            ob0[pl.ds(i * L, L)] = zeros16
            return 0
        lax.fori_loop(0, BLK // L, _z, 0)
        for j in range(S_TILE // BLK):
            pltpu.async_copy(ob0, acc.at[pl.ds(sub * S_TILE + j * BLK, BLK)],
                             ssc0)
        for j in range(S_TILE // BLK):
            pltpu.make_async_copy(
                ob0, acc.at[pl.ds(sub * S_TILE + j * BLK, BLK)],
                ssc0).wait()
        plsc.subcore_barrier()

        def _blk2(g, _):
            for par in range(2):
                b = g * 2 + par
                _wait_in(b, par)

                @pl.when(b >= 2)
                def _():
                    _wait_scat(par)

                def _vec(i, _):
                    for u in range(4):
                        off = i * (4 * L) + u * L
                        vi = idxb[par][pl.ds(off, L)]
                        vv = valb[par][pl.ds(off, L)]
                        local = vi - base
                        inr = plsc.bitcast(local, jnp.uint32) < jnp.uint32(C)
                        dummy = vi & DUMMY_MASK
                        locb[par][pl.ds(off, L)] = jnp.where(inr, local, dummy)
                        vout[par][pl.ds(off, L)] = jnp.where(inr, vv, 0.0)
                    return 0
                lax.fori_loop(0, BLK // (4 * L), _vec, 0)

                pltpu.async_copy(vout[par], acc.at[locb[par]], ssc[par],
                                 add=True)

                @pl.when(b + 2 < NBLK)
                def _():
                    _issue_in(b + 2, par)
            return 0
        lax.fori_loop(0, NBLK // 2, _blk2, 0)

        _wait_scat(0)
        _wait_scat(1)
        plsc.subcore_barrier()

        pltpu.sync_copy(acc.at[pl.ds(sub * S_TILE, S_TILE)],
                        out_hbm.at[pl.ds(base + sub * S_TILE, S_TILE)])
        plsc.subcore_barrier()


_scatter = pl.kernel(
    _body,
    out_type=jax.ShapeDtypeStruct((TOTAL,), jnp.float32),
    mesh=plsc.VectorSubcoreMesh(
        core_axis_name="c", subcore_axis_name="s", num_cores=NC,
        num_subcores=NS),
    compiler_params=pltpu.CompilerParams(needs_layout_passes=False),
    scratch_types=[
        pltpu.VMEM_SHARED((C,), jnp.float32),
        pltpu.VMEM((BLK,), jnp.int32),
        pltpu.VMEM((BLK,), jnp.int32),
        pltpu.VMEM((BLK,), jnp.float32),
        pltpu.VMEM((BLK,), jnp.float32),
        pltpu.VMEM((BLK,), jnp.int32),
        pltpu.VMEM((BLK,), jnp.int32),
        pltpu.VMEM((BLK,), jnp.float32),
        pltpu.VMEM((BLK,), jnp.float32),
        pltpu.SemaphoreType.DMA,
        pltpu.SemaphoreType.DMA,
        pltpu.SemaphoreType.DMA,
        pltpu.SemaphoreType.DMA,
    ],
)


@jax.jit
def kernel(updates, mask):
    upd = updates.reshape(-1)
    idx = mask.reshape(-1).astype(jnp.int32)
    out = _scatter(upd, idx)
    return out.reshape(B_, H_ * 2, W_ * 2, CH)


# speedup vs baseline: 7.2300x; 1.0041x over previous
import jax
import jax.numpy as jnp
from jax import lax
from jax.experimental import pallas as pl
from jax.experimental.pallas import tpu as pltpu
from jax.experimental.pallas import tpu_sc as plsc

B_, H_, W_, CH = 4, 128, 128, 96
N = B_ * H_ * W_ * CH
TOTAL = N * 4
NC, NS, L = 2, 16, 16

C = 1_572_864
DUM = 65_536
ACC = C + DUM
NPASS = 8
S_TILE = C // NS
Z_TILE = ACC // NS
BLK = 4_096
PER_TILE = N // NS
NBLK = PER_TILE // BLK
DUMMY_MASK = DUM - 1


def _body(upd_hbm, idx_hbm, out_hbm, acc,
          ib0, ib1, vb0, vb1, vb2, lb0, lb1,
          si0, si1, sv0, sv1, sv2, sc0, sc1):
    idxb = [ib0, ib1]
    valb = [vb0, vb1, vb2]
    locb = [lb0, lb1]
    sin = [si0, si1]
    svin = [sv0, sv1, sv2]
    ssc = [sc0, sc1]
    core = lax.axis_index("c")
    sub = lax.axis_index("s")
    tile_start = sub * PER_TILE
    zeros16 = jnp.zeros((L,), jnp.float32)

    def _issue_idx(b, p2):
        st = tile_start + b * BLK
        pltpu.async_copy(idx_hbm.at[pl.ds(st, BLK)], idxb[p2], sin[p2])

    def _wait_idx(b, p2):
        st = tile_start + b * BLK
        pltpu.make_async_copy(idx_hbm.at[pl.ds(st, BLK)], idxb[p2],
                              sin[p2]).wait()

    def _issue_val(b, p3):
        st = tile_start + b * BLK
        pltpu.async_copy(upd_hbm.at[pl.ds(st, BLK)], valb[p3], svin[p3])

    def _wait_val(b, p3):
        st = tile_start + b * BLK
        pltpu.make_async_copy(upd_hbm.at[pl.ds(st, BLK)], valb[p3],
                              svin[p3]).wait()

    def _wait_scat(p2, p3):
        pltpu.make_async_copy(valb[p3], acc.at[locb[p2]], ssc[p2]).wait()

    for p in range(NPASS):
        chunk = 2 * p + core
        base = chunk * C

        _issue_idx(0, 0)
        _issue_idx(1, 1)
        _issue_val(0, 0)

        def _z(i, _):
            vb2[pl.ds(i * L, L)] = zeros16
            return 0
        lax.fori_loop(0, BLK // L, _z, 0)
        for j in range(Z_TILE // BLK):
            pltpu.async_copy(vb2, acc.at[pl.ds(sub * Z_TILE + j * BLK, BLK)],
                             sc0)
        for j in range(Z_TILE // BLK):
            pltpu.make_async_copy(
                vb2, acc.at[pl.ds(sub * Z_TILE + j * BLK, BLK)],
                sc0).wait()
        plsc.subcore_barrier()

        def _blk6(g, _):
            for k in range(6):
                b6 = g * 6 + k
                p2 = k % 2
                p3 = k % 3

                _wait_idx(b6, p2)

                @pl.when(b6 >= 2)
                def _():
                    _wait_scat(p2, (k + 1) % 3)

                @pl.when(b6 + 1 < NBLK)
                def _():
                    _issue_val(b6 + 1, (k + 1) % 3)

                def _vec(i, _):
                    for u in range(4):
                        off = i * (4 * L) + u * L
                        vi = idxb[p2][pl.ds(off, L)]
                        local = vi - base
                        inr = plsc.bitcast(local, jnp.uint32) < jnp.uint32(C)
                        dummy = C + (vi & DUMMY_MASK)
                        locb[p2][pl.ds(off, L)] = jnp.where(inr, local, dummy)
                    return 0
                lax.fori_loop(0, BLK // (4 * L), _vec, 0)

                _wait_val(b6, p3)
                pltpu.async_copy(valb[p3], acc.at[locb[p2]], ssc[p2],
                                 add=True)

                @pl.when(b6 + 2 < NBLK)
                def _():
                    _issue_idx(b6 + 2, p2)
            return 0
        lax.fori_loop(0, NBLK // 6, _blk6, 0)

        _wait_scat(0, (NBLK - 2) % 3)
        _wait_scat(1, (NBLK - 1) % 3)
        plsc.subcore_barrier()

        pltpu.sync_copy(acc.at[pl.ds(sub * S_TILE, S_TILE)],
                        out_hbm.at[pl.ds(base + sub * S_TILE, S_TILE)])
        plsc.subcore_barrier()


_scatter = pl.kernel(
    _body,
    out_type=jax.ShapeDtypeStruct((TOTAL,), jnp.float32),
    mesh=plsc.VectorSubcoreMesh(
        core_axis_name="c", subcore_axis_name="s", num_cores=NC,
        num_subcores=NS),
    compiler_params=pltpu.CompilerParams(needs_layout_passes=False),
    scratch_types=[
        pltpu.VMEM_SHARED((ACC,), jnp.float32),
        pltpu.VMEM((BLK,), jnp.int32),
        pltpu.VMEM((BLK,), jnp.int32),
        pltpu.VMEM((BLK,), jnp.float32),
        pltpu.VMEM((BLK,), jnp.float32),
        pltpu.VMEM((BLK,), jnp.float32),
        pltpu.VMEM((BLK,), jnp.int32),
        pltpu.VMEM((BLK,), jnp.int32),
        pltpu.SemaphoreType.DMA,
        pltpu.SemaphoreType.DMA,
        pltpu.SemaphoreType.DMA,
        pltpu.SemaphoreType.DMA,
        pltpu.SemaphoreType.DMA,
        pltpu.SemaphoreType.DMA,
        pltpu.SemaphoreType.DMA,
    ],
)


@jax.jit
def kernel(updates, mask):
    upd = updates.reshape(-1)
    idx = mask.reshape(-1).astype(jnp.int32)
    out = _scatter(upd, idx)
    return out.reshape(B_, H_ * 2, W_ * 2, CH)
